# Initial kernel scaffold; baseline (speedup 1.0000x reference)
import jax
import jax.numpy as jnp
from jax.experimental import pallas as pl


def kernel(x, edge_index, vn_embed, Wl, bl, Wr, bn_g, bn_b, mlp_W1, mlp_b1, mlp_ln1_g, mlp_ln1_b, mlp_W2, mlp_b2, mlp_ln2_g, mlp_ln2_b):
    raise NotImplementedError("write your pallas kernel here")


# speedup vs baseline: 5.0840x; 5.0840x over previous
import functools

import jax
import jax.numpy as jnp
from jax import lax
from jax.experimental import pallas as pl
from jax.experimental.pallas import tpu as pltpu
from jax.experimental.pallas import tpu_sc as plsc

NC, NS = 2, 16
NW = NC * NS
DEGW = 16


def _slab_sizes(N):
    Z0 = (N // NS) // 8 * 8
    ZL = N - Z0 * (NS - 1)
    assert Z0 % 8 == 0 and ZL % 8 == 0 and ZL >= Z0
    return Z0, ZL


def _slab_copy(s, Z0, ZL, src_at, dst_at):
    @pl.when(s < NS - 1)
    def _():
        pltpu.sync_copy(src_at(Z0, s * Z0), dst_at(Z0, s * Z0))

    @pl.when(s == NS - 1)
    def _():
        pltpu.sync_copy(src_at(ZL, (NS - 1) * Z0), dst_at(ZL, (NS - 1) * Z0))


@functools.lru_cache(maxsize=None)
def _make_segsum(N, D, E):
    EW = E // NW
    K = 80
    CH = EW // K
    assert EW * NW == E and CH * K == EW and CH % 2 == 1
    Z0, ZL = _slab_sizes(N)
    n_pairs = (CH - 1) // 2

    mesh = plsc.VectorSubcoreMesh(core_axis_name="c", subcore_axis_name="s")

    del n_pairs
    out_type = [jax.ShapeDtypeStruct((NC, N, D), jnp.float32)]
    scratch = [
        pltpu.VMEM((CH, K), jnp.int32),
        pltpu.VMEM((CH, K), jnp.int32),
        pltpu.VMEM((K, D), jnp.float32),
        pltpu.VMEM_SHARED((N, D), jnp.float32),
        pltpu.SemaphoreType.DMA,
    ]

    def body(h_hbm, srcr, dstr, z_agg, agg_out, src_v, dst_v, rows_a,
             agg_sh, sem_a):
        c = lax.axis_index("c")
        s = lax.axis_index("s")
        w = s * NC + c
        pltpu.sync_copy(srcr.at[w], src_v)
        pltpu.sync_copy(dstr.at[w], dst_v)
        _slab_copy(s, Z0, ZL,
                   lambda n, o: z_agg.at[pl.ds(0, n)],
                   lambda n, o: agg_sh.at[pl.ds(o, n)])
        plsc.subcore_barrier()

        def chunk(j, carry):
            pltpu.async_copy(h_hbm.at[src_v.at[j]], rows_a, sem_a).wait()
            pltpu.sync_copy(rows_a, agg_sh.at[dst_v.at[j]], add=True)
            return carry

        lax.fori_loop(0, CH, chunk, 0)
        plsc.subcore_barrier()
        _slab_copy(s, Z0, ZL,
                   lambda n, o: agg_sh.at[pl.ds(o, n)],
                   lambda n, o: agg_out.at[c, pl.ds(o, n)])

    return pl.kernel(body, out_type=out_type, mesh=mesh,
                     scratch_types=scratch)


@functools.lru_cache(maxsize=None)
def _make_conv(N, D, H, BLK):
    grid = N // BLK
    assert grid * BLK == N

    def body(a0, a1, d0, d1, h, vn, wlt, blv, wrt, y, stats):
        i = pl.program_id(0)
        deg = d0[:, 0:1] + d1[:, 0:1]
        ssum = a0[...] + a1[...] + deg * vn[...]
        agg = ssum / jnp.maximum(deg, 1.0)
        hv = h[...] + vn[...]
        yv = (jnp.dot(agg, wlt[...], preferred_element_type=jnp.float32)
              + blv[...]
              + jnp.dot(hv, wrt[...], preferred_element_type=jnp.float32))
        y[...] = yv

        @pl.when(i == 0)
        def _():
            stats[...] = jnp.zeros_like(stats)

        stats[0:1, :] += jnp.sum(yv, axis=0, keepdims=True)
        stats[1:2, :] += jnp.sum(yv * yv, axis=0, keepdims=True)

    row = lambda i: (i, 0)
    fix = lambda i: (0, 0)
    return pl.pallas_call(
        body,
        grid=(grid,),
        in_specs=[
            pl.BlockSpec((BLK, D), row), pl.BlockSpec((BLK, D), row),
            pl.BlockSpec((BLK, D), row), pl.BlockSpec((BLK, D), row),
            pl.BlockSpec((BLK, D), row), pl.BlockSpec((1, D), fix),
            pl.BlockSpec((D, H), fix), pl.BlockSpec((1, H), fix),
            pl.BlockSpec((D, H), fix),
        ],
        out_specs=[pl.BlockSpec((BLK, H), row), pl.BlockSpec((8, H), fix)],
        out_shape=[jax.ShapeDtypeStruct((N, H), jnp.float32),
                   jax.ShapeDtypeStruct((8, H), jnp.float32)],
    )


@functools.lru_cache(maxsize=None)
def _make_bn(N, H, BLK, relu):
    grid = N // BLK
    inv_n = 1.0 / N

    def body(y, stats, g, b, hn, pooled):
        i = pl.program_id(0)
        mu = stats[0:1, :] * inv_n
        var = stats[1:2, :] * inv_n - mu * mu
        z = (y[...] - mu) * lax.rsqrt(var + 1e-5) * g[...] + b[...]
        if relu:
            z = jnp.maximum(z, 0.0)
        hn[...] = z

        @pl.when(i == 0)
        def _():
            pooled[...] = jnp.zeros_like(pooled)

        pooled[0:1, :] += jnp.sum(z, axis=0, keepdims=True)

    row = lambda i: (i, 0)
    fix = lambda i: (0, 0)
    return pl.pallas_call(
        body,
        grid=(grid,),
        in_specs=[
            pl.BlockSpec((BLK, H), row), pl.BlockSpec((8, H), fix),
            pl.BlockSpec((1, H), fix), pl.BlockSpec((1, H), fix),
        ],
        out_specs=[pl.BlockSpec((BLK, H), row), pl.BlockSpec((8, H), fix)],
        out_shape=[jax.ShapeDtypeStruct((N, H), jnp.float32),
                   jax.ShapeDtypeStruct((8, H), jnp.float32)],
    )


@functools.lru_cache(maxsize=None)
def _make_vn_mlp(H):
    H2 = 2 * H

    def ln(z, g, b):
        mu = jnp.mean(z, axis=-1, keepdims=True)
        zc = z - mu
        var = jnp.mean(zc * zc, axis=-1, keepdims=True)
        return zc * lax.rsqrt(var + 1e-5) * g + b

    def body(pooled, vn, w1t, b1, g1, be1, w2t, b2, g2, be2, out):
        t = pooled[0:1, :] + vn[...]
        z = jnp.dot(t, w1t[...], preferred_element_type=jnp.float32) + b1[...]
        z = jnp.maximum(z, 0.0)
        z = ln(z, g1[...], be1[...])
        z = jnp.dot(z, w2t[...], preferred_element_type=jnp.float32) + b2[...]
        z = jnp.maximum(z, 0.0)
        z = ln(z, g2[...], be2[...])
        out[...] = z

    return pl.pallas_call(
        body,
        out_shape=jax.ShapeDtypeStruct((1, H), jnp.float32),
    )


def kernel(x, edge_index, vn_embed, Wl, bl, Wr, bn_g, bn_b, mlp_W1, mlp_b1,
           mlp_ln1_g, mlp_ln1_b, mlp_W2, mlp_b2, mlp_ln2_g, mlp_ln2_b):
    N, D = x.shape
    E = edge_index.shape[1]
    L, H, _ = Wl.shape
    BLK = 1000

    src = edge_index[0].astype(jnp.int32).reshape(NW, E // NW // 80, 80)
    dst = edge_index[1].astype(jnp.int32).reshape(NW, E // NW // 80, 80)
    zl = N - ((N // NS) // 8 * 8) * (NS - 1)
    z_agg = jnp.zeros((zl, D), jnp.float32)
    ones_tab = jnp.ones((N, D), jnp.float32)

    WlT = jnp.swapaxes(Wl, 1, 2)
    WrT = jnp.swapaxes(Wr, 1, 2)
    W1T = jnp.swapaxes(mlp_W1, 1, 2)
    W2T = jnp.swapaxes(mlp_W2, 1, 2)

    segsum = _make_segsum(N, D, E)
    conv = _make_conv(N, D, H, BLK)
    bn_mid = _make_bn(N, H, BLK, True)
    bn_last = _make_bn(N, H, BLK, False)
    vn_mlp = _make_vn_mlp(H)

    h = x
    vn = vn_embed
    (degp,) = segsum(ones_tab, src, dst, z_agg)
    for l in range(L):
        (aggp,) = segsum(h, src, dst, z_agg)
        y, stats = conv(aggp[0], aggp[1], degp[0], degp[1], h, vn,
                        WlT[l], bl[l:l + 1], WrT[l])
        bn = bn_mid if l < L - 1 else bn_last
        h, pooled = bn(y, stats, bn_g[l:l + 1], bn_b[l:l + 1])
        if l < L - 1:
            vn = vn_mlp(pooled, vn, W1T[l], mlp_b1[l:l + 1],
                        mlp_ln1_g[l:l + 1], mlp_ln1_b[l:l + 1],
                        W2T[l], mlp_b2[l:l + 1],
                        mlp_ln2_g[l:l + 1], mlp_ln2_b[l:l + 1])
    return h


# speedup vs baseline: 5.9110x; 1.1627x over previous
import functools

import jax
import jax.numpy as jnp
from jax import lax
from jax.experimental import pallas as pl
from jax.experimental.pallas import tpu as pltpu
from jax.experimental.pallas import tpu_sc as plsc

NC, NS = 2, 16
NW = NC * NS
DEGW = 16


def _slab_sizes(N):
    Z0 = (N // NS) // 8 * 8
    ZL = N - Z0 * (NS - 1)
    assert Z0 % 8 == 0 and ZL % 8 == 0 and ZL >= Z0
    return Z0, ZL


def _slab_copy(s, Z0, ZL, src_at, dst_at):
    @pl.when(s < NS - 1)
    def _():
        pltpu.sync_copy(src_at(Z0, s * Z0), dst_at(Z0, s * Z0))

    @pl.when(s == NS - 1)
    def _():
        pltpu.sync_copy(src_at(ZL, (NS - 1) * Z0), dst_at(ZL, (NS - 1) * Z0))


@functools.lru_cache(maxsize=None)
def _make_segsum(N, D, E, gather=True):
    EW = E // NW
    K = 80
    CH = EW // K
    assert EW * NW == E and CH * K == EW
    Z0, ZL = _slab_sizes(N)

    mesh = plsc.VectorSubcoreMesh(core_axis_name="c", subcore_axis_name="s")

    out_type = [jax.ShapeDtypeStruct((NC, N, D), jnp.float32)]
    scratch = [
        pltpu.VMEM((CH, K), jnp.int32),
        pltpu.VMEM((CH, K), jnp.int32),
        pltpu.VMEM((K, D), jnp.float32),
        pltpu.VMEM_SHARED((N, D), jnp.float32),
        pltpu.SemaphoreType.DMA,
    ]

    def body(h_hbm, srcr, dstr, z_agg, agg_out, src_v, dst_v, rows_a,
             agg_sh, sem_a):
        c = lax.axis_index("c")
        s = lax.axis_index("s")
        w = s * NC + c
        if gather:
            pltpu.sync_copy(srcr.at[w], src_v)
        else:
            pltpu.sync_copy(h_hbm, rows_a)
        pltpu.sync_copy(dstr.at[w], dst_v)
        _slab_copy(s, Z0, ZL,
                   lambda n, o: z_agg.at[pl.ds(0, n)],
                   lambda n, o: agg_sh.at[pl.ds(o, n)])
        plsc.subcore_barrier()

        def chunk(j, carry):
            if gather:
                pltpu.async_copy(h_hbm.at[src_v.at[j]], rows_a, sem_a).wait()
            pltpu.sync_copy(rows_a, agg_sh.at[dst_v.at[j]], add=True)
            return carry

        lax.fori_loop(0, CH, chunk, 0)
        plsc.subcore_barrier()
        _slab_copy(s, Z0, ZL,
                   lambda n, o: agg_sh.at[pl.ds(o, n)],
                   lambda n, o: agg_out.at[c, pl.ds(o, n)])

    return pl.kernel(body, out_type=out_type, mesh=mesh,
                     scratch_types=scratch)


@functools.lru_cache(maxsize=None)
def _make_conv(N, D, H, BLK):
    grid = N // BLK
    assert grid * BLK == N

    def body(a0, a1, d0, d1, h, vn, wlt, blv, wrt, y, stats):
        i = pl.program_id(0)
        deg = d0[:, 0:1] + d1[:, 0:1]
        ssum = a0[...] + a1[...] + deg * vn[...]
        agg = ssum / jnp.maximum(deg, 1.0)
        hv = h[...] + vn[...]
        yv = (jnp.dot(agg, wlt[...], preferred_element_type=jnp.float32)
              + blv[...]
              + jnp.dot(hv, wrt[...], preferred_element_type=jnp.float32))
        y[...] = yv

        @pl.when(i == 0)
        def _():
            stats[...] = jnp.zeros_like(stats)

        stats[0:1, :] += jnp.sum(yv, axis=0, keepdims=True)
        stats[1:2, :] += jnp.sum(yv * yv, axis=0, keepdims=True)

    row = lambda i: (i, 0)
    fix = lambda i: (0, 0)
    return pl.pallas_call(
        body,
        grid=(grid,),
        in_specs=[
            pl.BlockSpec((BLK, D), row), pl.BlockSpec((BLK, D), row),
            pl.BlockSpec((BLK, D), row), pl.BlockSpec((BLK, D), row),
            pl.BlockSpec((BLK, D), row), pl.BlockSpec((1, D), fix),
            pl.BlockSpec((D, H), fix), pl.BlockSpec((1, H), fix),
            pl.BlockSpec((D, H), fix),
        ],
        out_specs=[pl.BlockSpec((BLK, H), row), pl.BlockSpec((8, H), fix)],
        out_shape=[jax.ShapeDtypeStruct((N, H), jnp.float32),
                   jax.ShapeDtypeStruct((8, H), jnp.float32)],
    )


@functools.lru_cache(maxsize=None)
def _make_bn(N, H, BLK, relu):
    grid = N // BLK
    inv_n = 1.0 / N

    def body(y, stats, g, b, hn, pooled):
        i = pl.program_id(0)
        mu = stats[0:1, :] * inv_n
        var = stats[1:2, :] * inv_n - mu * mu
        z = (y[...] - mu) * lax.rsqrt(var + 1e-5) * g[...] + b[...]
        if relu:
            z = jnp.maximum(z, 0.0)
        hn[...] = z

        @pl.when(i == 0)
        def _():
            pooled[...] = jnp.zeros_like(pooled)

        pooled[0:1, :] += jnp.sum(z, axis=0, keepdims=True)

    row = lambda i: (i, 0)
    fix = lambda i: (0, 0)
    return pl.pallas_call(
        body,
        grid=(grid,),
        in_specs=[
            pl.BlockSpec((BLK, H), row), pl.BlockSpec((8, H), fix),
            pl.BlockSpec((1, H), fix), pl.BlockSpec((1, H), fix),
        ],
        out_specs=[pl.BlockSpec((BLK, H), row), pl.BlockSpec((8, H), fix)],
        out_shape=[jax.ShapeDtypeStruct((N, H), jnp.float32),
                   jax.ShapeDtypeStruct((8, H), jnp.float32)],
    )


@functools.lru_cache(maxsize=None)
def _make_vn_mlp(H):
    H2 = 2 * H

    def ln(z, g, b):
        mu = jnp.mean(z, axis=-1, keepdims=True)
        zc = z - mu
        var = jnp.mean(zc * zc, axis=-1, keepdims=True)
        return zc * lax.rsqrt(var + 1e-5) * g + b

    def body(pooled, vn, w1t, b1, g1, be1, w2t, b2, g2, be2, out):
        t = pooled[0:1, :] + vn[...]
        z = jnp.dot(t, w1t[...], preferred_element_type=jnp.float32) + b1[...]
        z = jnp.maximum(z, 0.0)
        z = ln(z, g1[...], be1[...])
        z = jnp.dot(z, w2t[...], preferred_element_type=jnp.float32) + b2[...]
        z = jnp.maximum(z, 0.0)
        z = ln(z, g2[...], be2[...])
        out[...] = z

    return pl.pallas_call(
        body,
        out_shape=jax.ShapeDtypeStruct((1, H), jnp.float32),
    )


def kernel(x, edge_index, vn_embed, Wl, bl, Wr, bn_g, bn_b, mlp_W1, mlp_b1,
           mlp_ln1_g, mlp_ln1_b, mlp_W2, mlp_b2, mlp_ln2_g, mlp_ln2_b):
    N, D = x.shape
    E = edge_index.shape[1]
    L, H, _ = Wl.shape
    BLK = 1000

    src = edge_index[0].astype(jnp.int32).reshape(NW, E // NW // 80, 80)
    dst = edge_index[1].astype(jnp.int32).reshape(NW, E // NW // 80, 80)
    zl = N - ((N // NS) // 8 * 8) * (NS - 1)
    z_agg = jnp.zeros((zl, D), jnp.float32)
    ones_tab = jnp.ones((80, D), jnp.float32)

    WlT = jnp.swapaxes(Wl, 1, 2)
    WrT = jnp.swapaxes(Wr, 1, 2)
    W1T = jnp.swapaxes(mlp_W1, 1, 2)
    W2T = jnp.swapaxes(mlp_W2, 1, 2)

    segsum = _make_segsum(N, D, E)
    degsum = _make_segsum(N, D, E, gather=False)
    conv = _make_conv(N, D, H, BLK)
    bn_mid = _make_bn(N, H, BLK, True)
    bn_last = _make_bn(N, H, BLK, False)
    vn_mlp = _make_vn_mlp(H)

    h = x
    vn = vn_embed
    (degp,) = degsum(ones_tab, src, dst, z_agg)
    for l in range(L):
        (aggp,) = segsum(h, src, dst, z_agg)
        y, stats = conv(aggp[0], aggp[1], degp[0], degp[1], h, vn,
                        WlT[l], bl[l:l + 1], WrT[l])
        bn = bn_mid if l < L - 1 else bn_last
        h, pooled = bn(y, stats, bn_g[l:l + 1], bn_b[l:l + 1])
        if l < L - 1:
            vn = vn_mlp(pooled, vn, W1T[l], mlp_b1[l:l + 1],
                        mlp_ln1_g[l:l + 1], mlp_ln1_b[l:l + 1],
                        W2T[l], mlp_b2[l:l + 1],
                        mlp_ln2_g[l:l + 1], mlp_ln2_b[l:l + 1])
    return h
